# baseline (speedup 1.0000x reference)
import jax
import jax.numpy as jnp
from jax import lax
from jax.experimental import pallas as pl
from jax.experimental.pallas import tpu as pltpu

_NCB = 3


def _head_kernel(xs_hbm, g_ref, w_ref, b_ref, o_ref, xbuf, sems):
    t = pl.program_id(0)
    nt = pl.num_programs(0)
    ll = xs_hbm.shape[3]
    ps = xs_hbm.shape[0]
    bb = xbuf.shape[2]

    def copy(slot, tt, i):
        return pltpu.make_async_copy(
            xs_hbm.at[i, :, pl.ds(tt * _NCB, _NCB), ll - 1, :],
            xbuf.at[slot, i], sems.at[slot, i])

    @pl.when(t == 0)
    def _():
        for i in range(ps):
            copy(0, 0, i).start()
        for i in range(ps):
            copy(1, 1, i).start()

    for i in range(ps):
        copy(t % 2, t, i).wait()

    g = jnp.maximum(g_ref[...], 0.0)
    x = xbuf[t % 2]
    combs = []
    for k in range(_NCB):
        comb = x[0, :, k, :] * g[:, 0:1]
        for i in range(1, ps):
            comb = comb + x[i, :, k, :] * g[:, i:i + 1]
        combs.append((comb + 1e-9).T)
    combt = jnp.concatenate(combs, axis=1)

    @pl.when(t + 2 < nt)
    def _():
        for i in range(ps):
            copy(t % 2, t + 2, i).start()

    res = jax.lax.dot_general(
        w_ref[...], combt, (((1,), (0,)), ((), ())),
        preferred_element_type=jnp.float32)
    for k in range(_NCB):
        o_ref[k] = res[:, k * bb:(k + 1) * bb] + b_ref[...]


def kernel(xs, gates, W, b):
    ps, bb, cc, ll, dd = xs.shape
    pred = W.shape[0]
    b2 = b.reshape(pred, 1)
    grid = (cc // _NCB,)
    out_cpb = pl.pallas_call(
        _head_kernel,
        grid=grid,
        in_specs=[
            pl.BlockSpec(memory_space=pl.ANY),
            pl.BlockSpec((bb, ps), lambda t: (0, 0)),
            pl.BlockSpec((pred, dd), lambda t: (0, 0)),
            pl.BlockSpec((pred, 1), lambda t: (0, 0)),
        ],
        out_specs=pl.BlockSpec((_NCB, pred, bb), lambda t: (t, 0, 0)),
        out_shape=jax.ShapeDtypeStruct((cc, pred, bb), jnp.float32),
        scratch_shapes=[
            pltpu.VMEM((2, ps, bb, _NCB, dd), jnp.float32),
            pltpu.SemaphoreType.DMA((2, ps)),
        ],
    )(xs, gates, W, b2)
    return jnp.transpose(out_cpb, (2, 1, 0))


# speedup vs baseline: 1.0170x; 1.0170x over previous
import jax
import jax.numpy as jnp
from jax import lax
from jax.experimental import pallas as pl
from jax.experimental.pallas import tpu as pltpu

_NBUF = 3
_BSPLIT = 2


def _head_kernel(xs_hbm, g_ref, w_ref, b_ref, o_ref, xbuf, sems):
    c = pl.program_id(0)
    nc = pl.num_programs(0)
    ll = xs_hbm.shape[3]
    ps = xs_hbm.shape[0]
    bb = xbuf.shape[2]
    bh = bb // _BSPLIT

    def copy(slot, cc, i, h):
        return pltpu.make_async_copy(
            xs_hbm.at[i, pl.ds(h * bh, bh), cc, ll - 1, :],
            xbuf.at[slot, i, pl.ds(h * bh, bh), :],
            sems.at[slot, i, h])

    def start_all(slot, cc):
        for i in range(ps):
            for h in range(_BSPLIT):
                copy(slot, cc, i, h).start()

    @pl.when(c == 0)
    def _():
        for j in range(_NBUF - 1):
            start_all(j, j)

    for i in range(ps):
        for h in range(_BSPLIT):
            copy(c % _NBUF, c, i, h).wait()

    g = jnp.maximum(g_ref[...], 0.0)
    x = xbuf[c % _NBUF]
    comb = x[0] * g[:, 0:1]
    for i in range(1, ps):
        comb = comb + x[i] * g[:, i:i + 1]
    comb = comb + 1e-9

    @pl.when(c + _NBUF - 1 < nc)
    def _():
        start_all((c + _NBUF - 1) % _NBUF, c + _NBUF - 1)

    res = jax.lax.dot_general(
        w_ref[...], comb.T, (((1,), (0,)), ((), ())),
        preferred_element_type=jnp.float32)
    o_ref[0] = res + b_ref[...]


def kernel(xs, gates, W, b):
    ps, bb, cc, ll, dd = xs.shape
    pred = W.shape[0]
    b2 = b.reshape(pred, 1)
    grid = (cc,)
    out_cpb = pl.pallas_call(
        _head_kernel,
        grid=grid,
        in_specs=[
            pl.BlockSpec(memory_space=pl.ANY),
            pl.BlockSpec((bb, ps), lambda t: (0, 0)),
            pl.BlockSpec((pred, dd), lambda t: (0, 0)),
            pl.BlockSpec((pred, 1), lambda t: (0, 0)),
        ],
        out_specs=pl.BlockSpec((1, pred, bb), lambda t: (t, 0, 0)),
        out_shape=jax.ShapeDtypeStruct((cc, pred, bb), jnp.float32),
        scratch_shapes=[
            pltpu.VMEM((_NBUF, ps, bb, dd), jnp.float32),
            pltpu.SemaphoreType.DMA((_NBUF, ps, _BSPLIT)),
        ],
    )(xs, gates, W, b2)
    return jnp.transpose(out_cpb, (2, 1, 0))
